# baseline (speedup 1.0000x reference)
import functools

import numpy as np

import jax
import jax.numpy as jnp
from jax.experimental import pallas as pl
from jax.experimental.pallas import tpu as pltpu

_LANE = 128


def _se_flat_kernel(x_ref, mask_ref, w1a_ref, w1b_ref, b1_ref,
                    w2a_ref, w2b_ref, b2a_ref, b2b_ref, o_ref, *, bn):
    mask = mask_ref[...]
    w1a = w1a_ref[...]
    w1b = w1b_ref[...]
    for b in range(bn):
        xb = x_ref[b]
        xl = xb * mask
        xr = xb - xl
        z = jnp.dot(w1a, xl, preferred_element_type=jnp.float32)
        z = z + jnp.dot(w1b, xr, preferred_element_type=jnp.float32)
        z = jnp.sum(z, axis=1, keepdims=True) + b1_ref[...]
        h = jnp.maximum(z, 0.0)
        va = jnp.dot(w2a_ref[...], h, preferred_element_type=jnp.float32)
        vb = jnp.dot(w2b_ref[...], h, preferred_element_type=jnp.float32)
        sa = jnp.clip(va + b2a_ref[...] + 3.0, 0.0, 6.0) * (1.0 / 6.0)
        sb = jnp.clip(vb + b2b_ref[...] + 3.0, 0.0, 6.0) * (1.0 / 6.0)
        scale = sb + (sa - sb) * mask
        o_ref[b] = xb * scale


def kernel(x, w1, b1, w2, b2):
    N, C, H, W = x.shape
    HW = H * W
    Csq = w1.shape[0]
    flat = C * HW
    assert flat % _LANE == 0
    R = flat // _LANE

    r_idx = np.arange(R)
    ca = (r_idx * _LANE) // HW
    cb = (r_idx * _LANE + _LANE - 1) // HW
    lane = np.arange(_LANE)
    chan = (r_idx[:, None] * _LANE + lane[None, :]) // HW
    mask = (chan == ca[:, None]).astype(np.float32)

    inv = np.float32(1.0 / HW)
    w1a = w1[:, ca] * inv
    w1b = w1[:, cb] * inv
    w2a = w2[ca, :]
    w2b = w2[cb, :]
    b2a = b2[ca].reshape(R, 1)
    b2b = b2[cb].reshape(R, 1)
    b1c = b1.reshape(Csq, 1)
    mask_j = jnp.asarray(mask)

    x3 = x.reshape(N, R, _LANE)

    bn = 8
    while N % bn:
        bn //= 2
    db = x.dtype.itemsize
    cost = pl.CostEstimate(
        flops=2 * N * (2 * Csq * flat + 2 * R * Csq) + 3 * N * flat,
        transcendentals=0,
        bytes_accessed=2 * N * flat * db,
    )

    out3 = pl.pallas_call(
        functools.partial(_se_flat_kernel, bn=bn),
        out_shape=jax.ShapeDtypeStruct((N, R, _LANE), x.dtype),
        grid=(N // bn,),
        in_specs=[
            pl.BlockSpec((bn, R, _LANE), lambda n: (n, 0, 0)),
            pl.BlockSpec((R, _LANE), lambda n: (0, 0)),
            pl.BlockSpec((Csq, R), lambda n: (0, 0)),
            pl.BlockSpec((Csq, R), lambda n: (0, 0)),
            pl.BlockSpec((Csq, 1), lambda n: (0, 0)),
            pl.BlockSpec((R, Csq), lambda n: (0, 0)),
            pl.BlockSpec((R, Csq), lambda n: (0, 0)),
            pl.BlockSpec((R, 1), lambda n: (0, 0)),
            pl.BlockSpec((R, 1), lambda n: (0, 0)),
        ],
        out_specs=pl.BlockSpec((bn, R, _LANE), lambda n: (n, 0, 0)),
        compiler_params=pltpu.CompilerParams(
            dimension_semantics=("parallel",),
            vmem_limit_bytes=48 << 20,
        ),
        cost_estimate=cost,
    )(x3, mask_j, w1a, w1b, b1c, w2a, w2b, b2a, b2b)

    return out3.reshape(N, C, H, W)


# speedup vs baseline: 10.2571x; 10.2571x over previous
import functools

import jax
import jax.numpy as jnp
from jax.experimental import pallas as pl
from jax.experimental.pallas import tpu as pltpu


def _se_hwnc_kernel(x_ref, w1_ref, b1_ref, w2_ref, b2_ref, o_ref, *,
                    hw, unroll):
    S, C = x_ref.shape[1], x_ref.shape[2]

    def add_body(i, acc):
        for u in range(unroll):
            acc = acc + x_ref[i * unroll + u]
        return acc
    acc = jnp.zeros((S, C), jnp.float32)
    acc = jax.lax.fori_loop(0, hw // unroll, add_body, acc)
    for u in range(hw - hw % unroll, hw):
        acc = acc + x_ref[u]
    mean = acc * (1.0 / hw)

    z = jax.lax.dot_general(mean, w1_ref[...], (((1,), (1,)), ((), ())),
                            preferred_element_type=jnp.float32)
    h = jnp.maximum(z + b1_ref[...], 0.0)
    v = jax.lax.dot_general(h, w2_ref[...], (((1,), (1,)), ((), ())),
                            preferred_element_type=jnp.float32)
    s = jnp.clip(v + b2_ref[...] + 3.0, 0.0, 6.0) * (1.0 / 6.0)

    def mul_body(i, _):
        for u in range(unroll):
            j = i * unroll + u
            o_ref[j] = x_ref[j] * s
        return 0
    jax.lax.fori_loop(0, hw // unroll, mul_body, 0)
    for u in range(hw - hw % unroll, hw):
        o_ref[u] = x_ref[u] * s


def kernel(x, w1, b1, w2, b2):
    N, C, H, W = x.shape
    HW = H * W
    Csq = w1.shape[0]

    xt = x.transpose(2, 3, 0, 1).reshape(HW, N, C)
    b1r = b1.reshape(1, Csq)
    b2r = b2.reshape(1, C)

    S = 16
    while N % S:
        S //= 2
    db = x.dtype.itemsize
    cost = pl.CostEstimate(
        flops=2 * N * (2 * C * Csq) + 3 * N * C * HW,
        transcendentals=0,
        bytes_accessed=2 * N * C * HW * db,
    )

    out = pl.pallas_call(
        functools.partial(_se_hwnc_kernel, hw=HW, unroll=4),
        out_shape=jax.ShapeDtypeStruct((HW, N, C), x.dtype),
        grid=(N // S,),
        in_specs=[
            pl.BlockSpec((HW, S, C), lambda n: (0, n, 0)),
            pl.BlockSpec((Csq, C), lambda n: (0, 0)),
            pl.BlockSpec((1, Csq), lambda n: (0, 0)),
            pl.BlockSpec((C, Csq), lambda n: (0, 0)),
            pl.BlockSpec((1, C), lambda n: (0, 0)),
        ],
        out_specs=pl.BlockSpec((HW, S, C), lambda n: (0, n, 0)),
        compiler_params=pltpu.CompilerParams(
            dimension_semantics=("parallel",),
            vmem_limit_bytes=48 << 20,
        ),
        cost_estimate=cost,
    )(xt, w1, b1r, w2, b2r)

    return out.reshape(H, W, N, C).transpose(2, 3, 0, 1)


# speedup vs baseline: 11.0953x; 1.0817x over previous
import functools

import jax
import jax.numpy as jnp
from jax.experimental import pallas as pl
from jax.experimental.pallas import tpu as pltpu


def _se_hwnc_kernel(x_ref, w1_ref, b1_ref, w2_ref, b2_ref, o_ref, *,
                    hw, unroll):
    S, C = x_ref.shape[1], x_ref.shape[2]

    def add_body(i, acc):
        for u in range(unroll):
            acc = acc + x_ref[i * unroll + u]
        return acc
    acc = jnp.zeros((S, C), jnp.float32)
    acc = jax.lax.fori_loop(0, hw // unroll, add_body, acc)
    for u in range(hw - hw % unroll, hw):
        acc = acc + x_ref[u]
    mean = acc * (1.0 / hw)

    z = jax.lax.dot_general(mean, w1_ref[...], (((1,), (1,)), ((), ())),
                            preferred_element_type=jnp.float32)
    h = jnp.maximum(z + b1_ref[...], 0.0)
    v = jax.lax.dot_general(h, w2_ref[...], (((1,), (1,)), ((), ())),
                            preferred_element_type=jnp.float32)
    s = jnp.clip(v + b2_ref[...] + 3.0, 0.0, 6.0) * (1.0 / 6.0)

    def mul_body(i, _):
        for u in range(unroll):
            j = i * unroll + u
            o_ref[j] = x_ref[j] * s
        return 0
    jax.lax.fori_loop(0, hw // unroll, mul_body, 0)
    for u in range(hw - hw % unroll, hw):
        o_ref[u] = x_ref[u] * s


def kernel(x, w1, b1, w2, b2):
    N, C, H, W = x.shape
    HW = H * W
    Csq = w1.shape[0]

    xt = x.transpose(2, 3, 0, 1).reshape(HW, N, C)
    b1r = b1.reshape(1, Csq)
    b2r = b2.reshape(1, C)

    S = 32
    while N % S:
        S //= 2
    db = x.dtype.itemsize
    cost = pl.CostEstimate(
        flops=2 * N * (2 * C * Csq) + 3 * N * C * HW,
        transcendentals=0,
        bytes_accessed=2 * N * C * HW * db,
    )

    out = pl.pallas_call(
        functools.partial(_se_hwnc_kernel, hw=HW, unroll=4),
        out_shape=jax.ShapeDtypeStruct((HW, N, C), x.dtype),
        grid=(N // S,),
        in_specs=[
            pl.BlockSpec((HW, S, C), lambda n: (0, n, 0)),
            pl.BlockSpec((Csq, C), lambda n: (0, 0)),
            pl.BlockSpec((1, Csq), lambda n: (0, 0)),
            pl.BlockSpec((C, Csq), lambda n: (0, 0)),
            pl.BlockSpec((1, C), lambda n: (0, 0)),
        ],
        out_specs=pl.BlockSpec((HW, S, C), lambda n: (0, n, 0)),
        compiler_params=pltpu.CompilerParams(
            dimension_semantics=("parallel",),
            vmem_limit_bytes=48 << 20,
        ),
        cost_estimate=cost,
    )(xt, w1, b1r, w2, b2r)

    return out.reshape(H, W, N, C).transpose(2, 3, 0, 1)


# speedup vs baseline: 12.3119x; 1.1097x over previous
import functools

import jax
import jax.numpy as jnp
from jax.experimental import pallas as pl
from jax.experimental.pallas import tpu as pltpu


def _se_hwnc_kernel(x_ref, w1_ref, b1_ref, w2_ref, b2_ref, o_ref, *,
                    hw, unroll):
    S, C = x_ref.shape[1], x_ref.shape[2]

    def add_body(i, acc):
        for u in range(unroll):
            acc = acc + x_ref[i * unroll + u]
        return acc
    acc = jnp.zeros((S, C), jnp.float32)
    acc = jax.lax.fori_loop(0, hw // unroll, add_body, acc)
    for u in range(hw - hw % unroll, hw):
        acc = acc + x_ref[u]
    mean = acc * (1.0 / hw)

    z = jax.lax.dot_general(mean, w1_ref[...], (((1,), (1,)), ((), ())),
                            preferred_element_type=jnp.float32)
    h = jnp.maximum(z + b1_ref[...], 0.0)
    v = jax.lax.dot_general(h, w2_ref[...], (((1,), (1,)), ((), ())),
                            preferred_element_type=jnp.float32)
    s = jnp.clip(v + b2_ref[...] + 3.0, 0.0, 6.0) * (1.0 / 6.0)

    def mul_body(i, _):
        for u in range(unroll):
            j = i * unroll + u
            o_ref[j] = x_ref[j] * s
        return 0
    jax.lax.fori_loop(0, hw // unroll, mul_body, 0)
    for u in range(hw - hw % unroll, hw):
        o_ref[u] = x_ref[u] * s


def kernel(x, w1, b1, w2, b2):
    N, C, H, W = x.shape
    HW = H * W
    Csq = w1.shape[0]

    xt = x.transpose(2, 3, 0, 1).reshape(HW, N, C)
    b1r = b1.reshape(1, Csq)
    b2r = b2.reshape(1, C)

    S = 64
    while N % S:
        S //= 2
    db = x.dtype.itemsize
    cost = pl.CostEstimate(
        flops=2 * N * (2 * C * Csq) + 3 * N * C * HW,
        transcendentals=0,
        bytes_accessed=2 * N * C * HW * db,
    )

    out = pl.pallas_call(
        functools.partial(_se_hwnc_kernel, hw=HW, unroll=4),
        out_shape=jax.ShapeDtypeStruct((HW, N, C), x.dtype),
        grid=(N // S,),
        in_specs=[
            pl.BlockSpec((HW, S, C), lambda n: (0, n, 0)),
            pl.BlockSpec((Csq, C), lambda n: (0, 0)),
            pl.BlockSpec((1, Csq), lambda n: (0, 0)),
            pl.BlockSpec((C, Csq), lambda n: (0, 0)),
            pl.BlockSpec((1, C), lambda n: (0, 0)),
        ],
        out_specs=pl.BlockSpec((HW, S, C), lambda n: (0, n, 0)),
        compiler_params=pltpu.CompilerParams(
            dimension_semantics=("parallel",),
            vmem_limit_bytes=57 << 20,
        ),
        cost_estimate=cost,
    )(xt, w1, b1r, w2, b2r)

    return out.reshape(H, W, N, C).transpose(2, 3, 0, 1)


# speedup vs baseline: 12.5137x; 1.0164x over previous
import functools

import jax
import jax.numpy as jnp
from jax.experimental import pallas as pl
from jax.experimental.pallas import tpu as pltpu


def _se_hwnc_kernel(x_ref, w1_ref, b1_ref, w2_ref, b2_ref, o_ref, *,
                    hw, unroll):
    S, C = x_ref.shape[1], x_ref.shape[2]

    def add_body(i, acc):
        for u in range(unroll):
            acc = acc + x_ref[i * unroll + u]
        return acc
    acc = jnp.zeros((S, C), jnp.float32)
    acc = jax.lax.fori_loop(0, hw // unroll, add_body, acc)
    for u in range(hw - hw % unroll, hw):
        acc = acc + x_ref[u]
    mean = acc * (1.0 / hw)

    z = jax.lax.dot_general(mean, w1_ref[...], (((1,), (1,)), ((), ())),
                            preferred_element_type=jnp.float32)
    h = jnp.maximum(z + b1_ref[...], 0.0)
    v = jax.lax.dot_general(h, w2_ref[...], (((1,), (1,)), ((), ())),
                            preferred_element_type=jnp.float32)
    s = jnp.clip(v + b2_ref[...] + 3.0, 0.0, 6.0) * (1.0 / 6.0)

    def mul_body(i, _):
        for u in range(unroll):
            j = i * unroll + u
            o_ref[j] = x_ref[j] * s
        return 0
    jax.lax.fori_loop(0, hw // unroll, mul_body, 0)
    for u in range(hw - hw % unroll, hw):
        o_ref[u] = x_ref[u] * s


def kernel(x, w1, b1, w2, b2):
    N, C, H, W = x.shape
    HW = H * W
    Csq = w1.shape[0]

    xt = x.transpose(2, 3, 0, 1).reshape(HW, N, C)
    b1r = b1.reshape(1, Csq)
    b2r = b2.reshape(1, C)

    S = 64
    while N % S:
        S //= 2
    db = x.dtype.itemsize
    cost = pl.CostEstimate(
        flops=2 * N * (2 * C * Csq) + 3 * N * C * HW,
        transcendentals=0,
        bytes_accessed=2 * N * C * HW * db,
    )

    out = pl.pallas_call(
        functools.partial(_se_hwnc_kernel, hw=HW, unroll=14),
        out_shape=jax.ShapeDtypeStruct((HW, N, C), x.dtype),
        grid=(N // S,),
        in_specs=[
            pl.BlockSpec((HW, S, C), lambda n: (0, n, 0)),
            pl.BlockSpec((Csq, C), lambda n: (0, 0)),
            pl.BlockSpec((1, Csq), lambda n: (0, 0)),
            pl.BlockSpec((C, Csq), lambda n: (0, 0)),
            pl.BlockSpec((1, C), lambda n: (0, 0)),
        ],
        out_specs=pl.BlockSpec((HW, S, C), lambda n: (0, n, 0)),
        compiler_params=pltpu.CompilerParams(
            dimension_semantics=("parallel",),
            vmem_limit_bytes=57 << 20,
        ),
        cost_estimate=cost,
    )(xt, w1, b1r, w2, b2r)

    return out.reshape(H, W, N, C).transpose(2, 3, 0, 1)
